# Initial kernel scaffold; baseline (speedup 1.0000x reference)
import jax
import jax.numpy as jnp
from jax.experimental import pallas as pl


def kernel(u, target_ids, neighbor_ids, entity_table, user_table, W, a):
    raise NotImplementedError("write your pallas kernel here")


# speedup vs baseline: 1.2162x; 1.2162x over previous
import jax
import jax.numpy as jnp
from jax import lax
from jax.experimental import pallas as pl
from jax.experimental.pallas import tpu as pltpu
from jax.experimental.pallas import tpu_sc as plsc

B = 16384
K = 50
DIM = 16
NC = 2
NS = 16
NW = NC * NS
RPW = B // NW
C = 64
NCHUNK = RPW // C
NG = C // 16
NSUB = C * K // 128


def _rsqrt(q):
    i = plsc.bitcast(q, jnp.int32)
    y = plsc.bitcast(jnp.int32(0x5F3759DF) - (i >> 1), jnp.float32)
    for _ in range(3):
        y = y * (1.5 - 0.5 * q * y * y)
    return y


def _body(nb_hbm, tid_hbm, uid_hbm, et_hbm, ut_hbm, w_hbm, a_hbm, out_hbm,
          idx_v, nbr_v, tid_v, uid_v, trow_v, urow_v, w_v, a_v, out_v,
          sem):
    cid = lax.axis_index("c")
    sid = lax.axis_index("s")
    wid = sid * NC + cid

    pltpu.sync_copy(w_hbm, w_v)
    pltpu.sync_copy(a_hbm, a_v)
    arow = a_v[0, :]
    wrows = [w_v[i, :] for i in range(8)]
    v1 = jnp.zeros((16,), jnp.float32)
    v2 = jnp.zeros((16,), jnp.float32)
    for i in range(8):
        v1 = v1 + wrows[i] * arow[i]
        v2 = v2 + wrows[i] * arow[8 + i]
    v1s = [v1[d] for d in range(16)]
    v2s = [v2[d] for d in range(16)]

    iota = lax.iota(jnp.int32, 16)
    cds = [jnp.full((16,), d, jnp.int32) for d in range(16)]

    def chunk_body(c, carry):
        base_row = wid * RPW + c * C
        flat0 = wid * (RPW * K) + c * (C * K)
        pltpu.sync_copy(nb_hbm.at[pl.ds(flat0, C * K)], idx_v)
        pltpu.sync_copy(tid_hbm.at[pl.ds(base_row, C)], tid_v)
        pltpu.sync_copy(uid_hbm.at[pl.ds(base_row, C)], uid_v)
        cps = []
        for j in range(NSUB):
            cps.append(pltpu.async_copy(et_hbm.at[idx_v.at[pl.ds(j * 128, 128)]],
                                        nbr_v.at[pl.ds(j * 128, 128)], sem))
        cps.append(pltpu.async_copy(et_hbm.at[tid_v], trow_v, sem))
        cps.append(pltpu.async_copy(ut_hbm.at[uid_v], urow_v, sem))
        for cp in cps:
            cp.wait()

        def group_body(g, carry2):
            row16 = g * 16 + iota

            tcols = [plsc.load_gather(trow_v, [row16, cds[d]])
                     for d in range(16)]
            tq = tcols[0] * tcols[0]
            tp = tcols[0] * v1s[0]
            for d in range(1, 16):
                tq = tq + tcols[d] * tcols[d]
                tp = tp + tcols[d] * v1s[d]
            st = tp * jnp.minimum(1.0, _rsqrt(tq))

            row50 = row16 * K
            zero = jnp.zeros((16,), jnp.float32)

            def k_body(k, kc):
                ssum, acc = kc
                ridx = row50 + k
                cols = [plsc.load_gather(nbr_v, [ridx, cds[d]])
                        for d in range(16)]
                q = cols[0] * cols[0]
                p = cols[0] * v2s[0]
                for d in range(1, 16):
                    q = q + cols[d] * cols[d]
                    p = p + cols[d] * v2s[d]
                scl = jnp.minimum(1.0, _rsqrt(q))
                e = st + p * scl
                e = jnp.where(e >= 0.0, e, e * 0.2)
                w = jnp.exp(e)
                t = w * scl
                acc = tuple(acc[d] + t * cols[d] for d in range(16))
                return (ssum + w, acc)

            ssum, acc = lax.fori_loop(0, K, k_body, (zero, (zero,) * 16))

            ucols = [plsc.load_gather(urow_v, [row16, cds[d]])
                     for d in range(16)]
            uq = ucols[0] * ucols[0]
            for d in range(1, 16):
                uq = uq + ucols[d] * ucols[d]
            us = jnp.minimum(1.0, _rsqrt(uq))
            gs = [ucols[i] + ucols[8 + i] for i in range(8)]
            uv = zero
            for j in range(16):
                wv = gs[0] * wrows[0][j]
                for i in range(1, 8):
                    wv = wv + gs[i] * wrows[i][j]
                uv = uv + acc[j] * wv
            uv = uv * us / ssum
            logit = 1.0 / (1.0 + jnp.exp(-uv))
            out_v[pl.ds(g * 16, 16)] = logit
            return 0

        lax.fori_loop(0, NG, group_body, 0)
        pltpu.sync_copy(out_v, out_hbm.at[pl.ds(base_row, C)])
        return 0

    lax.fori_loop(0, NCHUNK, chunk_body, 0)


def kernel(u, target_ids, neighbor_ids, entity_table, user_table, W, a):
    nb2d = neighbor_ids.astype(jnp.int32).reshape(B * K)
    tids = target_ids.astype(jnp.int32)
    uids = u.astype(jnp.int32)
    mesh = plsc.VectorSubcoreMesh(core_axis_name="c", subcore_axis_name="s")
    fn = pl.kernel(
        _body,
        out_type=jax.ShapeDtypeStruct((B,), jnp.float32),
        mesh=mesh,
        scratch_types=[
            pltpu.VMEM((C * K,), jnp.int32),
            pltpu.VMEM((C * K, DIM), jnp.float32),
            pltpu.VMEM((C,), jnp.int32),
            pltpu.VMEM((C,), jnp.int32),
            pltpu.VMEM((C, DIM), jnp.float32),
            pltpu.VMEM((C, DIM), jnp.float32),
            pltpu.VMEM((8, DIM), jnp.float32),
            pltpu.VMEM((1, DIM), jnp.float32),
            pltpu.VMEM((C,), jnp.float32),
            pltpu.SemaphoreType.DMA,
        ],
        compiler_params=pltpu.CompilerParams(
            needs_layout_passes=False, use_tc_tiling_on_sc=False),
    )
    return fn(nb2d, tids, uids, entity_table, user_table, W, a)


# speedup vs baseline: 1.2791x; 1.0517x over previous
import jax
import jax.numpy as jnp
from jax import lax
from jax.experimental import pallas as pl
from jax.experimental.pallas import tpu as pltpu
from jax.experimental.pallas import tpu_sc as plsc

B = 16384
K = 50
DIM = 16
NC = 2
NS = 16
NW = NC * NS
RPW = B // NW
C = 64
NCHUNK = RPW // C
NG = C // 16
NSUB = C * K // 128
TROW = RPW // 128


def _rsqrt(q):
    i = plsc.bitcast(q, jnp.int32)
    y = plsc.bitcast(jnp.int32(0x5F3759DF) - (i >> 1), jnp.float32)
    for _ in range(3):
        y = y * (1.5 - 0.5 * q * y * y)
    return y


def _body(nb_hbm, tid_hbm, uid_hbm, et_hbm, ut_hbm, w_hbm, a_hbm, out_hbm,
          idx_v, nbr_v, tid_v, uid_v, trow_v, urow_v, w_v, a_v, out_v,
          sem):
    cid = lax.axis_index("c")
    sid = lax.axis_index("s")
    wid = sid * NC + cid

    pltpu.sync_copy(w_hbm, w_v)
    pltpu.sync_copy(a_hbm, a_v)
    arow = a_v[0, :]
    wrows = [w_v[i, :] for i in range(8)]
    v1 = jnp.zeros((16,), jnp.float32)
    v2 = jnp.zeros((16,), jnp.float32)
    for i in range(8):
        v1 = v1 + wrows[i] * arow[i]
        v2 = v2 + wrows[i] * arow[8 + i]
    v1s = [v1[d] for d in range(16)]
    v2s = [v2[d] for d in range(16)]

    iota = lax.iota(jnp.int32, 16)
    cds = [jnp.full((16,), d, jnp.int32) for d in range(16)]

    pltpu.sync_copy(nb_hbm.at[wid], idx_v)
    pltpu.sync_copy(tid_hbm.at[wid], tid_v)
    pltpu.sync_copy(uid_hbm.at[wid], uid_v)
    cp_t = pltpu.async_copy(et_hbm.at[tid_v], trow_v, sem)
    cp_u = pltpu.async_copy(ut_hbm.at[uid_v], urow_v, sem)
    cp_t.wait()
    cp_u.wait()

    def chunk_body(c, carry):
        cp = pltpu.async_copy(et_hbm.at[idx_v.at[pl.ds(c * (C * K), C * K)]],
                              nbr_v, sem)
        cp.wait()

        def group_body(g, carry2):
            row16 = c * C + g * 16 + iota

            tcols = [plsc.load_gather(trow_v, [row16, cds[d]])
                     for d in range(16)]
            tq = tcols[0] * tcols[0]
            tp = tcols[0] * v1s[0]
            for d in range(1, 16):
                tq = tq + tcols[d] * tcols[d]
                tp = tp + tcols[d] * v1s[d]
            st = tp * jnp.minimum(1.0, _rsqrt(tq))

            row50 = (g * 16 + iota) * K
            zero = jnp.zeros((16,), jnp.float32)

            def k_body(k, kc):
                ssum, acc = kc
                r = row50 + k
                cols = [plsc.load_gather(nbr_v, [r, cds[d]])
                        for d in range(16)]
                q = cols[0] * cols[0]
                p = cols[0] * v2s[0]
                for d in range(1, 16):
                    q = q + cols[d] * cols[d]
                    p = p + cols[d] * v2s[d]
                scl = jnp.minimum(1.0, _rsqrt(q))
                e = st + p * scl
                e = jnp.where(e >= 0.0, e, e * 0.2)
                w = jnp.exp(e)
                t = w * scl
                acc = tuple(acc[d] + t * cols[d] for d in range(16))
                return (ssum + w, acc)

            ssum, acc = lax.fori_loop(0, K, k_body, (zero, (zero,) * 16))

            ucols = [plsc.load_gather(urow_v, [row16, cds[d]])
                     for d in range(16)]
            uq = ucols[0] * ucols[0]
            for d in range(1, 16):
                uq = uq + ucols[d] * ucols[d]
            us = jnp.minimum(1.0, _rsqrt(uq))
            gs = [ucols[i] + ucols[8 + i] for i in range(8)]
            uv = zero
            for j in range(16):
                wv = gs[0] * wrows[0][j]
                for i in range(1, 8):
                    wv = wv + gs[i] * wrows[i][j]
                uv = uv + acc[j] * wv
            uv = uv * us / ssum
            logit = 1.0 / (1.0 + jnp.exp(-uv))
            out_v[pl.ds(c * C + g * 16, 16)] = logit
            return 0

        lax.fori_loop(0, NG, group_body, 0)
        return 0

    lax.fori_loop(0, NCHUNK, chunk_body, 0)
    pltpu.sync_copy(out_v, out_hbm.at[pl.ds(wid * RPW, RPW)])


def kernel(u, target_ids, neighbor_ids, entity_table, user_table, W, a):
    nb3d = neighbor_ids.astype(jnp.int32).reshape(NW, RPW * K)
    tids = target_ids.astype(jnp.int32).reshape(NW, RPW)
    uids = u.astype(jnp.int32).reshape(NW, RPW)
    mesh = plsc.VectorSubcoreMesh(core_axis_name="c", subcore_axis_name="s")
    fn = pl.kernel(
        _body,
        out_type=jax.ShapeDtypeStruct((B,), jnp.float32),
        mesh=mesh,
        scratch_types=[
            pltpu.VMEM((RPW * K,), jnp.int32),
            pltpu.VMEM((C * K, DIM), jnp.float32),
            pltpu.VMEM((RPW,), jnp.int32),
            pltpu.VMEM((RPW,), jnp.int32),
            pltpu.VMEM((RPW, DIM), jnp.float32),
            pltpu.VMEM((RPW, DIM), jnp.float32),
            pltpu.VMEM((8, DIM), jnp.float32),
            pltpu.VMEM((1, DIM), jnp.float32),
            pltpu.VMEM((RPW,), jnp.float32),
            pltpu.SemaphoreType.DMA,
        ],
        compiler_params=pltpu.CompilerParams(
            needs_layout_passes=False, use_tc_tiling_on_sc=False),
    )
    return fn(nb3d, tids, uids, entity_table, user_table, W, a)


# speedup vs baseline: 1.6919x; 1.3228x over previous
import jax
import jax.numpy as jnp
from jax import lax
from jax.experimental import pallas as pl
from jax.experimental.pallas import tpu as pltpu
from jax.experimental.pallas import tpu_sc as plsc

B = 16384
K = 50
DIM = 16
NC = 2
NS = 16
NW = NC * NS
RPW = B // NW
C = 64
NCHUNK = RPW // C
NG = C // 16

_PARAMS = dict(
    compiler_params=pltpu.CompilerParams(
        needs_layout_passes=False, use_tc_tiling_on_sc=False),
)


def _rsqrt(q):
    i = plsc.bitcast(q, jnp.int32)
    y = plsc.bitcast(jnp.int32(0x5F3759DF) - (i >> 1), jnp.float32)
    for _ in range(3):
        y = y * (1.5 - 0.5 * q * y * y)
    return y


def _stage_wa(w_hbm, a_hbm, w_v, a_v):
    pltpu.sync_copy(w_hbm, w_v)
    pltpu.sync_copy(a_hbm, a_v)
    arow = a_v[0, :]
    wrows = [w_v[i, :] for i in range(8)]
    return arow, wrows


def _body_a(nb_hbm, tid_hbm, et_hbm, w_hbm, a_hbm, y_hbm,
            idx_v, nbr_v, tid_v, trow_v, w_v, a_v, y_v, sem):
    cid = lax.axis_index("c")
    sid = lax.axis_index("s")
    wid = sid * NC + cid

    arow, wrows = _stage_wa(w_hbm, a_hbm, w_v, a_v)
    v1 = jnp.zeros((16,), jnp.float32)
    v2 = jnp.zeros((16,), jnp.float32)
    for i in range(8):
        v1 = v1 + wrows[i] * arow[i]
        v2 = v2 + wrows[i] * arow[8 + i]
    v1s = [v1[d] for d in range(16)]
    v2s = [v2[d] for d in range(16)]

    iota = lax.iota(jnp.int32, 16)
    cds = [jnp.full((16,), d, jnp.int32) for d in range(16)]

    pltpu.sync_copy(nb_hbm.at[wid], idx_v)
    pltpu.sync_copy(tid_hbm.at[wid], tid_v)
    pltpu.async_copy(et_hbm.at[tid_v], trow_v, sem).wait()

    def chunk_body(c, carry):
        cp = pltpu.async_copy(et_hbm.at[idx_v.at[pl.ds(c * (C * K), C * K)]],
                              nbr_v, sem)
        cp.wait()

        def group_body(g, carry2):
            row16 = c * C + g * 16 + iota

            tcols = [plsc.load_gather(trow_v, [row16, cds[d]])
                     for d in range(16)]
            tq = tcols[0] * tcols[0]
            tp = tcols[0] * v1s[0]
            for d in range(1, 16):
                tq = tq + tcols[d] * tcols[d]
                tp = tp + tcols[d] * v1s[d]
            st = tp * jnp.minimum(1.0, _rsqrt(tq))

            row50 = (g * 16 + iota) * K
            zero = jnp.zeros((16,), jnp.float32)

            def k_body(k, kc):
                ssum, acc = kc
                r = row50 + k
                cols = [plsc.load_gather(nbr_v, [r, cds[d]])
                        for d in range(16)]
                q = cols[0] * cols[0]
                p = cols[0] * v2s[0]
                for d in range(1, 16):
                    q = q + cols[d] * cols[d]
                    p = p + cols[d] * v2s[d]
                scl = jnp.minimum(1.0, _rsqrt(q))
                e = st + p * scl
                e = jnp.where(e >= 0.0, e, e * 0.2)
                w = jnp.exp(e)
                t = w * scl
                acc = tuple(acc[d] + t * cols[d] for d in range(16))
                return (ssum + w, acc)

            ssum, acc = lax.fori_loop(0, K, k_body, (zero, (zero,) * 16))

            inv = 1.0 / ssum
            for d in range(16):
                plsc.store_scatter(y_v, [row16, cds[d]], acc[d] * inv)
            return 0

        lax.fori_loop(0, NG, group_body, 0)
        return 0

    lax.fori_loop(0, NCHUNK, chunk_body, 0)
    pltpu.sync_copy(y_v, y_hbm.at[pl.ds(wid * RPW, RPW)])


def _body_b(y_hbm, uid_hbm, ut_hbm, w_hbm, a_hbm, out_hbm,
            uid_v, urow_v, y_v, w_v, a_v, out_v, sem):
    cid = lax.axis_index("c")
    sid = lax.axis_index("s")
    wid = sid * NC + cid

    _, wrows = _stage_wa(w_hbm, a_hbm, w_v, a_v)
    iota = lax.iota(jnp.int32, 16)
    cds = [jnp.full((16,), d, jnp.int32) for d in range(16)]

    pltpu.sync_copy(uid_hbm.at[wid], uid_v)
    pltpu.sync_copy(y_hbm.at[pl.ds(wid * RPW, RPW)], y_v)
    pltpu.async_copy(ut_hbm.at[uid_v], urow_v, sem).wait()

    def group_body(g, carry):
        row16 = g * 16 + iota
        ucols = [plsc.load_gather(urow_v, [row16, cds[d]])
                 for d in range(16)]
        uq = ucols[0] * ucols[0]
        for d in range(1, 16):
            uq = uq + ucols[d] * ucols[d]
        us = jnp.minimum(1.0, _rsqrt(uq))
        gs = [ucols[i] + ucols[8 + i] for i in range(8)]
        uv = jnp.zeros((16,), jnp.float32)
        for j in range(16):
            wv = gs[0] * wrows[0][j]
            for i in range(1, 8):
                wv = wv + gs[i] * wrows[i][j]
            yj = plsc.load_gather(y_v, [row16, cds[j]])
            uv = uv + yj * wv
        uv = uv * us
        logit = 1.0 / (1.0 + jnp.exp(-uv))
        out_v[pl.ds(g * 16, 16)] = logit
        return 0

    lax.fori_loop(0, RPW // 16, group_body, 0)
    pltpu.sync_copy(out_v, out_hbm.at[pl.ds(wid * RPW, RPW)])


def kernel(u, target_ids, neighbor_ids, entity_table, user_table, W, a):
    nb2d = neighbor_ids.astype(jnp.int32).reshape(NW, RPW * K)
    tids = target_ids.astype(jnp.int32).reshape(NW, RPW)
    uids = u.astype(jnp.int32).reshape(NW, RPW)
    mesh = plsc.VectorSubcoreMesh(core_axis_name="c", subcore_axis_name="s")
    fn_a = pl.kernel(
        _body_a,
        out_type=jax.ShapeDtypeStruct((B, DIM), jnp.float32),
        mesh=mesh,
        scratch_types=[
            pltpu.VMEM((RPW * K,), jnp.int32),
            pltpu.VMEM((C * K, DIM), jnp.float32),
            pltpu.VMEM((RPW,), jnp.int32),
            pltpu.VMEM((RPW, DIM), jnp.float32),
            pltpu.VMEM((8, DIM), jnp.float32),
            pltpu.VMEM((1, DIM), jnp.float32),
            pltpu.VMEM((RPW, DIM), jnp.float32),
            pltpu.SemaphoreType.DMA,
        ],
        **_PARAMS,
    )
    y = fn_a(nb2d, tids, entity_table, W, a)
    fn_b = pl.kernel(
        _body_b,
        out_type=jax.ShapeDtypeStruct((B,), jnp.float32),
        mesh=mesh,
        scratch_types=[
            pltpu.VMEM((RPW,), jnp.int32),
            pltpu.VMEM((RPW, DIM), jnp.float32),
            pltpu.VMEM((RPW, DIM), jnp.float32),
            pltpu.VMEM((8, DIM), jnp.float32),
            pltpu.VMEM((1, DIM), jnp.float32),
            pltpu.VMEM((RPW,), jnp.float32),
            pltpu.SemaphoreType.DMA,
        ],
        **_PARAMS,
    )
    return fn_b(y, uids, user_table, W, a)
